# initial kernel scaffold (unmeasured)
import jax
import jax.numpy as jnp
from jax import lax
from jax.experimental import pallas as pl
from jax.experimental.pallas import tpu as pltpu


def kernel(
    x,
):
    def body(*refs):
        pass

    out_shape = jax.ShapeDtypeStruct(..., jnp.float32)
    return pl.pallas_call(body, out_shape=out_shape)(...)


# baseline (device time: 224071 ns/iter reference)
import jax
import jax.numpy as jnp
from jax import lax
from jax.experimental import pallas as pl
from jax.experimental.pallas import tpu as pltpu

M, N = 8192, 1024
CHUNK = 1024
NCH = M // CHUNK


def kernel(x):
    def body(x_hbm, out_hbm, send_buf, recv_buf, stage,
             copy_sems, send_sem, recv_sem, out_sem):
        my_x = lax.axis_index("x")
        my_y = lax.axis_index("y")
        my_z = lax.axis_index("z")
        peer = (1 - my_x, my_y, my_z)

        barrier_sem = pltpu.get_barrier_semaphore()
        pl.semaphore_signal(
            barrier_sem, inc=1, device_id=peer,
            device_id_type=pl.DeviceIdType.MESH,
        )
        pl.semaphore_wait(barrier_sem, 1)

        for c in range(NCH):
            slot = c % 2
            rows = pl.ds(c * CHUNK, CHUNK)
            cp = pltpu.make_async_copy(
                x_hbm.at[rows, :], stage.at[slot], copy_sems.at[slot])
            cp.start()
            cp.wait()
            send_buf[rows, :] = stage[slot].astype(jnp.bfloat16)

        rdma = pltpu.make_async_remote_copy(
            src_ref=send_buf,
            dst_ref=recv_buf,
            send_sem=send_sem,
            recv_sem=recv_sem,
            device_id=peer,
            device_id_type=pl.DeviceIdType.MESH,
        )
        rdma.start()
        rdma.wait()

        send_buf[...] = send_buf[...] + recv_buf[...]
        out_cp = pltpu.make_async_copy(send_buf, out_hbm, out_sem)
        out_cp.start()
        out_cp.wait()

    return pl.pallas_call(
        body,
        out_shape=jax.ShapeDtypeStruct((M, N), jnp.bfloat16),
        in_specs=[pl.BlockSpec(memory_space=pl.ANY)],
        out_specs=pl.BlockSpec(memory_space=pl.ANY),
        scratch_shapes=[
            pltpu.VMEM((M, N), jnp.bfloat16),
            pltpu.VMEM((M, N), jnp.bfloat16),
            pltpu.VMEM((2, CHUNK, N), jnp.float32),
            pltpu.SemaphoreType.DMA((2,)),
            pltpu.SemaphoreType.DMA,
            pltpu.SemaphoreType.DMA,
            pltpu.SemaphoreType.DMA,
        ],
        compiler_params=pltpu.CompilerParams(
            collective_id=0, vmem_limit_bytes=60 * 1024 * 1024),
    )(x)


# device time: 133716 ns/iter; 1.6757x vs baseline; 1.6757x over previous
import jax
import jax.numpy as jnp
from jax import lax
from jax.experimental import pallas as pl
from jax.experimental.pallas import tpu as pltpu

M, N = 8192, 1024
QROWS = M // 4
CH = 512
KQ = QROWS // CH


def kernel(x):
    def body(x_hbm, out_hbm, loc32, rem, sendq, out_stage,
             in_sems, bulk_sems, send_sems, x_recv, y_recv, za_recv,
             zb_recv, out_sems):
        my_x = lax.axis_index("x")
        my_y = lax.axis_index("y")
        my_z = lax.axis_index("z")
        px = (1 - my_x, my_y, my_z)
        py = (my_x, 1 - my_y, my_z)
        pz = (my_x, my_y, 1 - my_z)

        q_own = 2 * my_y + my_z
        q_y = 2 * (1 - my_y) + my_z
        q_za = 2 * my_y + (1 - my_z)
        q_zb = 2 * (1 - my_y) + (1 - my_z)

        def rows(q, k=0, nrows=CH):
            return pl.ds(q * QROWS + k * CH, nrows)

        barrier_sem = pltpu.get_barrier_semaphore()
        for nbr in (px, py, pz):
            pl.semaphore_signal(
                barrier_sem, inc=1, device_id=nbr,
                device_id_type=pl.DeviceIdType.MESH,
            )
        pl.semaphore_wait(barrier_sem, 3)

        bulk_cps = []
        for i, q in enumerate((q_y, q_za, q_zb)):
            cp = pltpu.make_async_copy(
                x_hbm.at[rows(q, 0, QROWS), :],
                loc32.at[rows(q, 0, QROWS), :],
                bulk_sems.at[i],
            )
            cp.start()
            bulk_cps.append(cp)

        in_cps = []
        for k in range(KQ):
            cp = pltpu.make_async_copy(
                x_hbm.at[rows(q_own, k), :],
                loc32.at[rows(q_own, k), :],
                in_sems.at[k],
            )
            cp.start()
            in_cps.append(cp)

        x_rdmas = []
        for k in range(KQ):
            in_cps[k].wait()
            sendq[pl.ds(k * CH, CH), :] = loc32[rows(q_own, k), :].astype(
                jnp.bfloat16)
            rdma = pltpu.make_async_remote_copy(
                src_ref=sendq.at[pl.ds(k * CH, CH), :],
                dst_ref=rem.at[rows(q_own, k), :],
                send_sem=in_sems.at[k],
                recv_sem=x_recv.at[k],
                device_id=px,
                device_id_type=pl.DeviceIdType.MESH,
            )
            rdma.start()
            x_rdmas.append(rdma)

        y_rdmas, za_rdmas = [], []
        for k in range(KQ):
            x_rdmas[k].wait_recv()
            ry = pltpu.make_async_remote_copy(
                src_ref=rem.at[rows(q_own, k), :],
                dst_ref=rem.at[rows(q_own, k), :],
                send_sem=send_sems.at[0],
                recv_sem=y_recv.at[k],
                device_id=py,
                device_id_type=pl.DeviceIdType.MESH,
            )
            ry.start()
            y_rdmas.append(ry)
            rza = pltpu.make_async_remote_copy(
                src_ref=rem.at[rows(q_own, k), :],
                dst_ref=rem.at[rows(q_own, k), :],
                send_sem=send_sems.at[1],
                recv_sem=za_recv.at[k],
                device_id=pz,
                device_id_type=pl.DeviceIdType.MESH,
            )
            rza.start()
            za_rdmas.append(rza)

        zb_rdmas = []
        for k in range(KQ):
            ry_in = pltpu.make_async_remote_copy(
                src_ref=rem.at[rows(q_y, k), :],
                dst_ref=rem.at[rows(q_y, k), :],
                send_sem=send_sems.at[0],
                recv_sem=y_recv.at[k],
                device_id=py,
                device_id_type=pl.DeviceIdType.MESH,
            )
            ry_in.wait_recv()
            rzb = pltpu.make_async_remote_copy(
                src_ref=rem.at[rows(q_y, k), :],
                dst_ref=rem.at[rows(q_y, k), :],
                send_sem=send_sems.at[2],
                recv_sem=zb_recv.at[k],
                device_id=pz,
                device_id_type=pl.DeviceIdType.MESH,
            )
            rzb.start()
            zb_rdmas.append(rzb)

        out_cps = [None, None]
        emit_count = [0]

        def emit(q, k):
            i = emit_count[0]
            slot = i % 2
            if out_cps[slot] is not None:
                out_cps[slot].wait()
            out_stage[slot] = (
                loc32[rows(q, k), :].astype(jnp.bfloat16) + rem[rows(q, k), :]
            )
            cp = pltpu.make_async_copy(
                out_stage.at[slot], out_hbm.at[rows(q, k), :],
                out_sems.at[slot],
            )
            cp.start()
            out_cps[slot] = cp
            emit_count[0] += 1

        bulk_cps[0].wait()
        for k in range(KQ):
            emit(q_own, k)
        for k in range(KQ):
            emit(q_y, k)

        bulk_cps[1].wait()
        for k in range(KQ):
            za_rdmas[k].wait_recv()
            emit(q_za, k)
        bulk_cps[2].wait()
        for k in range(KQ):
            zb_rdmas[k].wait_recv()
            emit(q_zb, k)

        for cp in out_cps:
            if cp is not None:
                cp.wait()
        for k in range(KQ):
            x_rdmas[k].wait_send()
            y_rdmas[k].wait_send()
            za_rdmas[k].wait_send()
            zb_rdmas[k].wait_send()

    return pl.pallas_call(
        body,
        out_shape=jax.ShapeDtypeStruct((M, N), jnp.bfloat16),
        in_specs=[pl.BlockSpec(memory_space=pl.ANY)],
        out_specs=pl.BlockSpec(memory_space=pl.ANY),
        scratch_shapes=[
            pltpu.VMEM((M, N), jnp.float32),
            pltpu.VMEM((M, N), jnp.bfloat16),
            pltpu.VMEM((QROWS, N), jnp.bfloat16),
            pltpu.VMEM((2, CH, N), jnp.bfloat16),
            pltpu.SemaphoreType.DMA((KQ,)),
            pltpu.SemaphoreType.DMA((3,)),
            pltpu.SemaphoreType.DMA((3,)),
            pltpu.SemaphoreType.DMA((KQ,)),
            pltpu.SemaphoreType.DMA((KQ,)),
            pltpu.SemaphoreType.DMA((KQ,)),
            pltpu.SemaphoreType.DMA((KQ,)),
            pltpu.SemaphoreType.DMA((2,)),
        ],
        compiler_params=pltpu.CompilerParams(
            collective_id=0, vmem_limit_bytes=60 * 1024 * 1024),
    )(x)


# device time: 125411 ns/iter; 1.7867x vs baseline; 1.0662x over previous
import jax
import jax.numpy as jnp
from jax import lax
from jax.experimental import pallas as pl
from jax.experimental.pallas import tpu as pltpu

M, N = 8192, 1024
QROWS = M // 4
CH = 512
KQ = QROWS // CH


def kernel(x):
    def body(x_hbm, out_hbm, loc32, rem, sendq, out_stage,
             in_sems, bulk_sems, send_sems, x_recv, y_recv, za_recv,
             zb_recv, out_sems):
        my_x = lax.axis_index("x")
        my_y = lax.axis_index("y")
        my_z = lax.axis_index("z")
        px = (1 - my_x, my_y, my_z)
        py = (my_x, 1 - my_y, my_z)
        pz = (my_x, my_y, 1 - my_z)

        q_own = 2 * my_y + my_z
        q_y = 2 * (1 - my_y) + my_z
        q_za = 2 * my_y + (1 - my_z)
        q_zb = 2 * (1 - my_y) + (1 - my_z)

        def rows(q, k=0, nrows=CH):
            return pl.ds(q * QROWS + k * CH, nrows)

        barrier_sem = pltpu.get_barrier_semaphore()
        for nbr in (px, py, pz):
            pl.semaphore_signal(
                barrier_sem, inc=1, device_id=nbr,
                device_id_type=pl.DeviceIdType.MESH,
            )
        pl.semaphore_wait(barrier_sem, 3)

        in_cps = []
        for k in range(KQ):
            cp = pltpu.make_async_copy(
                x_hbm.at[rows(q_own, k), :],
                loc32.at[rows(q_own, k), :],
                in_sems.at[k],
            )
            cp.start()
            in_cps.append(cp)

        x_rdmas = []
        for k in range(KQ):
            in_cps[k].wait()
            sendq[pl.ds(k * CH, CH), :] = loc32[rows(q_own, k), :].astype(
                jnp.bfloat16)
            rdma = pltpu.make_async_remote_copy(
                src_ref=sendq.at[pl.ds(k * CH, CH), :],
                dst_ref=rem.at[rows(q_own, k), :],
                send_sem=in_sems.at[k],
                recv_sem=x_recv.at[k],
                device_id=px,
                device_id_type=pl.DeviceIdType.MESH,
            )
            rdma.start()
            x_rdmas.append(rdma)

        bulk_cps = []
        for i, q in enumerate((q_y, q_za, q_zb)):
            cp = pltpu.make_async_copy(
                x_hbm.at[rows(q, 0, QROWS), :],
                loc32.at[rows(q, 0, QROWS), :],
                bulk_sems.at[i],
            )
            cp.start()
            bulk_cps.append(cp)

        y_rdmas, za_rdmas = [], []
        for k in range(KQ):
            x_rdmas[k].wait_recv()
            ry = pltpu.make_async_remote_copy(
                src_ref=rem.at[rows(q_own, k), :],
                dst_ref=rem.at[rows(q_own, k), :],
                send_sem=send_sems.at[0],
                recv_sem=y_recv.at[k],
                device_id=py,
                device_id_type=pl.DeviceIdType.MESH,
            )
            ry.start()
            y_rdmas.append(ry)
            rza = pltpu.make_async_remote_copy(
                src_ref=rem.at[rows(q_own, k), :],
                dst_ref=rem.at[rows(q_own, k), :],
                send_sem=send_sems.at[1],
                recv_sem=za_recv.at[k],
                device_id=pz,
                device_id_type=pl.DeviceIdType.MESH,
            )
            rza.start()
            za_rdmas.append(rza)

        zb_rdmas = []
        for k in range(KQ):
            ry_in = pltpu.make_async_remote_copy(
                src_ref=rem.at[rows(q_y, k), :],
                dst_ref=rem.at[rows(q_y, k), :],
                send_sem=send_sems.at[0],
                recv_sem=y_recv.at[k],
                device_id=py,
                device_id_type=pl.DeviceIdType.MESH,
            )
            ry_in.wait_recv()
            rzb = pltpu.make_async_remote_copy(
                src_ref=rem.at[rows(q_y, k), :],
                dst_ref=rem.at[rows(q_y, k), :],
                send_sem=send_sems.at[2],
                recv_sem=zb_recv.at[k],
                device_id=pz,
                device_id_type=pl.DeviceIdType.MESH,
            )
            rzb.start()
            zb_rdmas.append(rzb)

        out_cps = [None, None]
        emit_count = [0]

        def emit(q, k):
            i = emit_count[0]
            slot = i % 2
            if out_cps[slot] is not None:
                out_cps[slot].wait()
            out_stage[slot] = (
                loc32[rows(q, k), :].astype(jnp.bfloat16) + rem[rows(q, k), :]
            )
            cp = pltpu.make_async_copy(
                out_stage.at[slot], out_hbm.at[rows(q, k), :],
                out_sems.at[slot],
            )
            cp.start()
            out_cps[slot] = cp
            emit_count[0] += 1

        bulk_cps[0].wait()
        for k in range(KQ):
            emit(q_own, k)
        for k in range(KQ):
            emit(q_y, k)

        bulk_cps[1].wait()
        for k in range(KQ):
            za_rdmas[k].wait_recv()
            emit(q_za, k)
        bulk_cps[2].wait()
        for k in range(KQ):
            zb_rdmas[k].wait_recv()
            emit(q_zb, k)

        for cp in out_cps:
            if cp is not None:
                cp.wait()
        for k in range(KQ):
            x_rdmas[k].wait_send()
            y_rdmas[k].wait_send()
            za_rdmas[k].wait_send()
            zb_rdmas[k].wait_send()

    return pl.pallas_call(
        body,
        out_shape=jax.ShapeDtypeStruct((M, N), jnp.bfloat16),
        in_specs=[pl.BlockSpec(memory_space=pl.ANY)],
        out_specs=pl.BlockSpec(memory_space=pl.ANY),
        scratch_shapes=[
            pltpu.VMEM((M, N), jnp.float32),
            pltpu.VMEM((M, N), jnp.bfloat16),
            pltpu.VMEM((QROWS, N), jnp.bfloat16),
            pltpu.VMEM((2, CH, N), jnp.bfloat16),
            pltpu.SemaphoreType.DMA((KQ,)),
            pltpu.SemaphoreType.DMA((3,)),
            pltpu.SemaphoreType.DMA((3,)),
            pltpu.SemaphoreType.DMA((KQ,)),
            pltpu.SemaphoreType.DMA((KQ,)),
            pltpu.SemaphoreType.DMA((KQ,)),
            pltpu.SemaphoreType.DMA((KQ,)),
            pltpu.SemaphoreType.DMA((2,)),
        ],
        compiler_params=pltpu.CompilerParams(
            collective_id=0, vmem_limit_bytes=60 * 1024 * 1024),
    )(x)


# device time: 103108 ns/iter; 2.1732x vs baseline; 1.2163x over previous
import jax
import jax.numpy as jnp
from jax import lax
from jax.experimental import pallas as pl
from jax.experimental.pallas import tpu as pltpu

M, N = 8192, 1024
QROWS = M // 4
CH = 512
KQ = QROWS // CH
KZ = KQ - 1


def kernel(x):
    def body(x_hbm, out_hbm, loc32, rem, sendq, out_stage,
             in_sems, bulk_sems, send_sems, xe_send, x_recv, y_recv,
             za_recv, zb_recv, xe_recv, out_sems):
        my_x = lax.axis_index("x")
        my_y = lax.axis_index("y")
        my_z = lax.axis_index("z")
        px = (1 - my_x, my_y, my_z)
        py = (my_x, 1 - my_y, my_z)
        pz = (my_x, my_y, 1 - my_z)

        q_own = 2 * my_y + my_z
        q_y = 2 * (1 - my_y) + my_z
        q_za = 2 * my_y + (1 - my_z)
        q_zb = 2 * (1 - my_y) + (1 - my_z)

        def rows(q, k=0, nrows=CH):
            return pl.ds(q * QROWS + k * CH, nrows)

        barrier_sem = pltpu.get_barrier_semaphore()
        for nbr in (px, py, pz):
            pl.semaphore_signal(
                barrier_sem, inc=1, device_id=nbr,
                device_id_type=pl.DeviceIdType.MESH,
            )
        pl.semaphore_wait(barrier_sem, 3)

        in_cps = []
        for k in range(KQ):
            cp = pltpu.make_async_copy(
                x_hbm.at[rows(q_own, k), :],
                loc32.at[rows(q_own, k), :],
                in_sems.at[k],
            )
            cp.start()
            in_cps.append(cp)

        x_rdmas = []
        for k in range(KQ):
            in_cps[k].wait()
            sendq[pl.ds(k * CH, CH), :] = loc32[rows(q_own, k), :].astype(
                jnp.bfloat16)
            rdma = pltpu.make_async_remote_copy(
                src_ref=sendq.at[pl.ds(k * CH, CH), :],
                dst_ref=rem.at[rows(q_own, k), :],
                send_sem=in_sems.at[k],
                recv_sem=x_recv.at[k],
                device_id=px,
                device_id_type=pl.DeviceIdType.MESH,
            )
            rdma.start()
            x_rdmas.append(rdma)

        bulk_cps = []
        for i, q in enumerate((q_y, q_za, q_zb)):
            cp = pltpu.make_async_copy(
                x_hbm.at[rows(q, 0, QROWS), :],
                loc32.at[rows(q, 0, QROWS), :],
                bulk_sems.at[i],
            )
            cp.start()
            bulk_cps.append(cp)

        xe_rdmas = []
        for i, q in enumerate((q_za, q_zb)):
            bulk_cps[1 + i].wait()
            slot = pl.ds((KQ + i) * CH, CH)
            sendq[slot, :] = loc32[rows(q, KZ), :].astype(jnp.bfloat16)
            rdma = pltpu.make_async_remote_copy(
                src_ref=sendq.at[slot, :],
                dst_ref=rem.at[rows(q, KZ), :],
                send_sem=xe_send.at[i],
                recv_sem=xe_recv.at[i],
                device_id=px,
                device_id_type=pl.DeviceIdType.MESH,
            )
            rdma.start()
            xe_rdmas.append(rdma)

        y_rdmas, za_rdmas = [], []
        for k in range(KQ):
            x_rdmas[k].wait_recv()
            ry = pltpu.make_async_remote_copy(
                src_ref=rem.at[rows(q_own, k), :],
                dst_ref=rem.at[rows(q_own, k), :],
                send_sem=send_sems.at[0],
                recv_sem=y_recv.at[k],
                device_id=py,
                device_id_type=pl.DeviceIdType.MESH,
            )
            ry.start()
            y_rdmas.append(ry)
            if k < KZ:
                rza = pltpu.make_async_remote_copy(
                    src_ref=rem.at[rows(q_own, k), :],
                    dst_ref=rem.at[rows(q_own, k), :],
                    send_sem=send_sems.at[1],
                    recv_sem=za_recv.at[k],
                    device_id=pz,
                    device_id_type=pl.DeviceIdType.MESH,
                )
                rza.start()
                za_rdmas.append(rza)

        zb_rdmas = []
        for k in range(KQ):
            ry_in = pltpu.make_async_remote_copy(
                src_ref=rem.at[rows(q_y, k), :],
                dst_ref=rem.at[rows(q_y, k), :],
                send_sem=send_sems.at[0],
                recv_sem=y_recv.at[k],
                device_id=py,
                device_id_type=pl.DeviceIdType.MESH,
            )
            ry_in.wait_recv()
            if k < KZ:
                rzb = pltpu.make_async_remote_copy(
                    src_ref=rem.at[rows(q_y, k), :],
                    dst_ref=rem.at[rows(q_y, k), :],
                    send_sem=send_sems.at[2],
                    recv_sem=zb_recv.at[k],
                    device_id=pz,
                    device_id_type=pl.DeviceIdType.MESH,
                )
                rzb.start()
                zb_rdmas.append(rzb)

        out_cps = [None, None]
        emit_count = [0]

        def emit(q, k):
            i = emit_count[0]
            slot = i % 2
            if out_cps[slot] is not None:
                out_cps[slot].wait()
            out_stage[slot] = (
                loc32[rows(q, k), :].astype(jnp.bfloat16) + rem[rows(q, k), :]
            )
            cp = pltpu.make_async_copy(
                out_stage.at[slot], out_hbm.at[rows(q, k), :],
                out_sems.at[slot],
            )
            cp.start()
            out_cps[slot] = cp
            emit_count[0] += 1

        bulk_cps[0].wait()
        for k in range(KQ):
            emit(q_own, k)
        for k in range(KQ):
            emit(q_y, k)

        for k in range(KZ):
            za_rdmas[k].wait_recv()
            emit(q_za, k)
        xe_rdmas[0].wait_recv()
        emit(q_za, KZ)
        for k in range(KZ):
            zb_rdmas[k].wait_recv()
            emit(q_zb, k)
        xe_rdmas[1].wait_recv()
        emit(q_zb, KZ)

        for cp in out_cps:
            if cp is not None:
                cp.wait()
        for k in range(KQ):
            x_rdmas[k].wait_send()
            y_rdmas[k].wait_send()
        for k in range(KZ):
            za_rdmas[k].wait_send()
            zb_rdmas[k].wait_send()
        xe_rdmas[0].wait_send()
        xe_rdmas[1].wait_send()

    return pl.pallas_call(
        body,
        out_shape=jax.ShapeDtypeStruct((M, N), jnp.bfloat16),
        in_specs=[pl.BlockSpec(memory_space=pl.ANY)],
        out_specs=pl.BlockSpec(memory_space=pl.ANY),
        scratch_shapes=[
            pltpu.VMEM((M, N), jnp.float32),
            pltpu.VMEM((M, N), jnp.bfloat16),
            pltpu.VMEM((QROWS + 2 * CH, N), jnp.bfloat16),
            pltpu.VMEM((2, CH, N), jnp.bfloat16),
            pltpu.SemaphoreType.DMA((KQ,)),
            pltpu.SemaphoreType.DMA((3,)),
            pltpu.SemaphoreType.DMA((3,)),
            pltpu.SemaphoreType.DMA((2,)),
            pltpu.SemaphoreType.DMA((KQ,)),
            pltpu.SemaphoreType.DMA((KQ,)),
            pltpu.SemaphoreType.DMA((KZ,)),
            pltpu.SemaphoreType.DMA((KZ,)),
            pltpu.SemaphoreType.DMA((2,)),
            pltpu.SemaphoreType.DMA((2,)),
        ],
        compiler_params=pltpu.CompilerParams(
            collective_id=0, vmem_limit_bytes=60 * 1024 * 1024),
    )(x)
